# baseline (speedup 1.0000x reference)
import functools

import jax
import jax.numpy as jnp
from jax import lax
from jax.experimental import pallas as pl
from jax.experimental.pallas import tpu as pltpu
from jax.experimental.pallas import tpu_sc as plsc

T = 2048
D = 2048
FF = 768
E = 8
LANES = 128
BM = 256
P = T * 2 + E * BM
NB = P // BM
NC = 2
NS = 16
NW = NC * NS
_SC_MESH = dict(core_axis_name="c", subcore_axis_name="s",
                num_cores=NC, num_subcores=NS)


def _router_body(x_ref, gw_ref, idx_ref, w1_ref, w2_ref, cnt_ref):
    x = x_ref[...]
    gw = gw_ref[...]
    logits = lax.dot_general(x.astype(jnp.bfloat16), gw.astype(jnp.bfloat16),
                             (((1,), (1,)), ((), ())),
                             preferred_element_type=jnp.float32)
    col = lax.broadcasted_iota(jnp.int32, (T, LANES), 1)
    lm = jnp.where(col < E, logits, jnp.float32(-1e30))
    m = jnp.max(lm, axis=1, keepdims=True)
    ex = jnp.exp(lm - m)
    p = ex / jnp.sum(ex, axis=1, keepdims=True)
    w1 = jnp.max(p, axis=1, keepdims=True)
    e1 = jnp.min(jnp.where((p == w1) & (col < E), col, LANES - 1),
                 axis=1, keepdims=True)
    oh1 = col == e1
    p2 = jnp.where(oh1 | (col >= E), jnp.float32(-1.0), p)
    w2 = jnp.max(p2, axis=1, keepdims=True)
    e2 = jnp.min(jnp.where(p2 == w2, col, LANES - 1), axis=1, keepdims=True)
    oh2 = col == e2
    den = w1 + w2
    w1n = w1 / den
    w2n = w2 / den
    oh = (oh1 | oh2).astype(jnp.float32)
    ri = lax.broadcasted_iota(jnp.int32, (T, T), 0)
    ci = lax.broadcasted_iota(jnp.int32, (T, T), 1)
    tri = (ci < ri).astype(jnp.float32)
    pos = lax.dot_general(tri, oh, (((1,), (0,)), ((), ())),
                          precision=lax.Precision.HIGHEST)
    counts = jnp.sum(oh, axis=0, keepdims=True).astype(jnp.int32)
    pc = ((counts + (BM - 1)) >> 8) << 8
    ii = lax.broadcasted_iota(jnp.int32, (LANES, LANES), 0)
    jj = lax.broadcasted_iota(jnp.int32, (LANES, LANES), 1)
    upper = (ii < jj).astype(jnp.float32)
    seg = lax.dot_general(pc.astype(jnp.float32), upper,
                          (((1,), (0,)), ((), ())),
                          precision=lax.Precision.HIGHEST).astype(jnp.int32)
    rmat = seg + pos.astype(jnp.int32)
    r1 = jnp.sum(jnp.where(oh1, rmat, 0), axis=1, keepdims=True)
    r2 = jnp.sum(jnp.where(oh2, rmat, 0), axis=1, keepdims=True)
    idx_ref[...] = jnp.where(col == 0, r1, jnp.where(col == 1, r2, 0))
    w1_ref[...] = jnp.broadcast_to(w1n, (T, LANES))
    w2_ref[...] = jnp.broadcast_to(w2n, (T, LANES))
    cnt_ref[...] = counts


def _router(x, gwp, interpret=False):
    return pl.pallas_call(
        _router_body,
        out_shape=(
            jax.ShapeDtypeStruct((T, LANES), jnp.int32),
            jax.ShapeDtypeStruct((T, LANES), jnp.float32),
            jax.ShapeDtypeStruct((T, LANES), jnp.float32),
            jax.ShapeDtypeStruct((1, LANES), jnp.int32),
        ),
        interpret=interpret,
    )(x, gwp)


def _mlp_body(be_ref, bv_ref, xs_ref, wrow_ref, wg_ref, wu_ref, wd_ref,
              out_ref):
    b = pl.program_id(0)

    @pl.when(bv_ref[b] != 0)
    def _():
        xb = xs_ref[...].astype(jnp.bfloat16)
        wg = wg_ref[0]
        wu = wu_ref[0]
        wd = wd_ref[0]
        g = lax.dot_general(xb, wg, (((1,), (1,)), ((), ())),
                            preferred_element_type=jnp.float32)
        u = lax.dot_general(xb, wu, (((1,), (1,)), ((), ())),
                            preferred_element_type=jnp.float32)
        h = g * jax.lax.logistic(g) * u
        y = lax.dot_general(h.astype(jnp.bfloat16), wd,
                            (((1,), (1,)), ((), ())),
                            preferred_element_type=jnp.float32)
        out_ref[...] = y * wrow_ref[...][:, 0:1]


def _mlp(xs, wrow, Wg, Wu, Wd, bexpert, bvalid, interpret=False):
    grid_spec = pltpu.PrefetchScalarGridSpec(
        num_scalar_prefetch=2,
        grid=(NB,),
        in_specs=[
            pl.BlockSpec((BM, D), lambda b, be, bv: (b, 0)),
            pl.BlockSpec((BM, LANES), lambda b, be, bv: (b, 0)),
            pl.BlockSpec((1, FF, D), lambda b, be, bv: (be[b], 0, 0)),
            pl.BlockSpec((1, FF, D), lambda b, be, bv: (be[b], 0, 0)),
            pl.BlockSpec((1, D, FF), lambda b, be, bv: (be[b], 0, 0)),
        ],
        out_specs=pl.BlockSpec((BM, D), lambda b, be, bv: (b, 0)),
    )
    return pl.pallas_call(
        _mlp_body,
        grid_spec=grid_spec,
        out_shape=jax.ShapeDtypeStruct((P, D), jnp.float32),
        interpret=interpret,
    )(bexpert, bvalid, xs, wrow, Wg, Wu, Wd)


CH_D = 32


def _dispatch_body(x_hbm, r1_hbm, r2_hbm, w1r_hbm, w2r_hbm, xs_hbm, wrow_hbm,
                   idx1_v, idx2_v, rows_v, w1r_v, w2r_v, sem):
    wid = lax.axis_index("s") * NC + lax.axis_index("c")
    tpw = T // NW
    for it in range(tpw // CH_D):
        base = wid * tpw + it * CH_D
        pltpu.sync_copy(x_hbm.at[pl.ds(base, CH_D)], rows_v)
        pltpu.sync_copy(r1_hbm.at[pl.ds(base, CH_D)], idx1_v)
        pltpu.sync_copy(r2_hbm.at[pl.ds(base, CH_D)], idx2_v)
        pltpu.sync_copy(w1r_hbm.at[pl.ds(base, CH_D)], w1r_v)
        pltpu.sync_copy(w2r_hbm.at[pl.ds(base, CH_D)], w2r_v)
        c1 = pltpu.async_copy(rows_v, xs_hbm.at[idx1_v], sem)
        c2 = pltpu.async_copy(rows_v, xs_hbm.at[idx2_v], sem)
        c3 = pltpu.async_copy(w1r_v, wrow_hbm.at[idx1_v], sem)
        c4 = pltpu.async_copy(w2r_v, wrow_hbm.at[idx2_v], sem)
        c1.wait()
        c2.wait()
        c3.wait()
        c4.wait()


def _dispatch(x, r1, r2, w1rep, w2rep):
    return pl.kernel(
        _dispatch_body,
        out_type=(
            jax.ShapeDtypeStruct((P, D), jnp.float32),
            jax.ShapeDtypeStruct((P, LANES), jnp.float32),
        ),
        mesh=plsc.VectorSubcoreMesh(**_SC_MESH),
        scratch_types=[
            pltpu.VMEM((CH_D,), jnp.int32),
            pltpu.VMEM((CH_D,), jnp.int32),
            pltpu.VMEM((CH_D, D), jnp.float32),
            pltpu.VMEM((CH_D, LANES), jnp.float32),
            pltpu.VMEM((CH_D, LANES), jnp.float32),
            pltpu.SemaphoreType.DMA,
        ],
    )(x, r1, r2, w1rep, w2rep)


CH_C = 16


def _combine_body(ys_hbm, r1_hbm, r2_hbm, out_hbm,
                  idx1_v, idx2_v, buf1_v, buf2_v, sem):
    wid = lax.axis_index("s") * NC + lax.axis_index("c")
    tpw = T // NW
    for it in range(tpw // CH_C):
        base = wid * tpw + it * CH_C
        pltpu.sync_copy(r1_hbm.at[pl.ds(base, CH_C)], idx1_v)
        pltpu.sync_copy(r2_hbm.at[pl.ds(base, CH_C)], idx2_v)
        c1 = pltpu.async_copy(ys_hbm.at[idx1_v], buf1_v, sem)
        c2 = pltpu.async_copy(ys_hbm.at[idx2_v], buf2_v, sem)
        c1.wait()
        c2.wait()

        def row_body(i, _):
            for j in range(D // 16):
                a = buf1_v[i, pl.ds(j * 16, 16)]
                b = buf2_v[i, pl.ds(j * 16, 16)]
                buf1_v[i, pl.ds(j * 16, 16)] = a + b
            return 0

        lax.fori_loop(0, CH_C, row_body, 0)
        pltpu.sync_copy(buf1_v, out_hbm.at[pl.ds(base, CH_C)])


def _combine(ys, r1, r2):
    return pl.kernel(
        _combine_body,
        out_type=jax.ShapeDtypeStruct((T, D), jnp.float32),
        mesh=plsc.VectorSubcoreMesh(**_SC_MESH),
        scratch_types=[
            pltpu.VMEM((CH_C,), jnp.int32),
            pltpu.VMEM((CH_C,), jnp.int32),
            pltpu.VMEM((CH_C, D), jnp.float32),
            pltpu.VMEM((CH_C, D), jnp.float32),
            pltpu.SemaphoreType.DMA,
        ],
    )(ys, r1, r2)


def _block_map(counts8):
    pc = ((counts8 + (BM - 1)) // BM) * BM
    ends = jnp.cumsum(pc) // BM
    b = jnp.arange(NB, dtype=jnp.int32)
    bexpert = jnp.minimum(
        jnp.sum(ends[None, :] <= b[:, None], axis=1).astype(jnp.int32), E - 1)
    bvalid = (b < ends[-1]).astype(jnp.int32)
    return bexpert, bvalid


def kernel(hidden_states, gate_w, Wg, Wu, Wd):
    bsz, seq, d = hidden_states.shape
    x = hidden_states.reshape(-1, d)
    gwp = jnp.zeros((LANES, D), jnp.float32).at[:E].set(gate_w)
    idx, w1rep, w2rep, cnt = _router(x, gwp)
    r1 = idx[:, 0]
    r2 = idx[:, 1]
    bexpert, bvalid = _block_map(cnt[0, :E])
    xs, wrow = _dispatch(x, r1, r2, w1rep, w2rep)
    ys = _mlp(xs, wrow, Wg.astype(jnp.bfloat16), Wu.astype(jnp.bfloat16),
              Wd.astype(jnp.bfloat16), bexpert, bvalid)
    out = _combine(ys, r1, r2)
    return out.reshape(bsz, seq, d)


# speedup vs baseline: 1.1977x; 1.1977x over previous
import functools

import jax
import jax.numpy as jnp
from jax import lax
from jax.experimental import pallas as pl
from jax.experimental.pallas import tpu as pltpu
from jax.experimental.pallas import tpu_sc as plsc

T = 2048
D = 2048
FF = 768
E = 8
LANES = 128
BM = 256
P = T * 2 + E * BM
NB = P // BM
NC = 2
NS = 16
NW = NC * NS
_SC_MESH = dict(core_axis_name="c", subcore_axis_name="s",
                num_cores=NC, num_subcores=NS)


def _router_body(x_ref, gw_ref, idx_ref, w1_ref, w2_ref, cnt_ref):
    x = x_ref[...]
    gw = gw_ref[...]
    logits = lax.dot_general(x.astype(jnp.bfloat16), gw.astype(jnp.bfloat16),
                             (((1,), (1,)), ((), ())),
                             preferred_element_type=jnp.float32)
    col = lax.broadcasted_iota(jnp.int32, (T, LANES), 1)
    lm = jnp.where(col < E, logits, jnp.float32(-1e30))
    m = jnp.max(lm, axis=1, keepdims=True)
    ex = jnp.exp(lm - m)
    p = ex / jnp.sum(ex, axis=1, keepdims=True)
    w1 = jnp.max(p, axis=1, keepdims=True)
    e1 = jnp.min(jnp.where((p == w1) & (col < E), col, LANES - 1),
                 axis=1, keepdims=True)
    oh1 = col == e1
    p2 = jnp.where(oh1 | (col >= E), jnp.float32(-1.0), p)
    w2 = jnp.max(p2, axis=1, keepdims=True)
    e2 = jnp.min(jnp.where(p2 == w2, col, LANES - 1), axis=1, keepdims=True)
    oh2 = col == e2
    den = w1 + w2
    w1n = w1 / den
    w2n = w2 / den
    oh = (oh1 | oh2).astype(jnp.float32)
    ri = lax.broadcasted_iota(jnp.int32, (T, T), 0)
    ci = lax.broadcasted_iota(jnp.int32, (T, T), 1)
    tri = (ci < ri).astype(jnp.float32)
    pos = lax.dot_general(tri, oh, (((1,), (0,)), ((), ())),
                          precision=lax.Precision.HIGHEST)
    counts = jnp.sum(oh, axis=0, keepdims=True).astype(jnp.int32)
    pc = ((counts + (BM - 1)) >> 8) << 8
    ii = lax.broadcasted_iota(jnp.int32, (LANES, LANES), 0)
    jj = lax.broadcasted_iota(jnp.int32, (LANES, LANES), 1)
    upper = (ii < jj).astype(jnp.float32)
    seg = lax.dot_general(pc.astype(jnp.float32), upper,
                          (((1,), (0,)), ((), ())),
                          precision=lax.Precision.HIGHEST).astype(jnp.int32)
    rmat = seg + pos.astype(jnp.int32)
    r1 = jnp.sum(jnp.where(oh1, rmat, 0), axis=1, keepdims=True)
    r2 = jnp.sum(jnp.where(oh2, rmat, 0), axis=1, keepdims=True)
    idx_ref[...] = jnp.where(col == 0, r1, jnp.where(col == 1, r2, 0))
    w1_ref[...] = jnp.broadcast_to(w1n, (T, LANES))
    w2_ref[...] = jnp.broadcast_to(w2n, (T, LANES))
    cnt_ref[...] = counts


def _router(x, gwp, interpret=False):
    return pl.pallas_call(
        _router_body,
        out_shape=(
            jax.ShapeDtypeStruct((T, LANES), jnp.int32),
            jax.ShapeDtypeStruct((T, LANES), jnp.float32),
            jax.ShapeDtypeStruct((T, LANES), jnp.float32),
            jax.ShapeDtypeStruct((1, LANES), jnp.int32),
        ),
        interpret=interpret,
    )(x, gwp)


def _mlp_body(be_ref, bv_ref, xs_ref, wrow_ref, wg_ref, wu_ref, wd_ref,
              out_ref):
    b = pl.program_id(0)

    @pl.when(bv_ref[b] != 0)
    def _():
        xb = xs_ref[...].astype(jnp.bfloat16)
        wg = wg_ref[0].astype(jnp.bfloat16)
        wu = wu_ref[0].astype(jnp.bfloat16)
        wd = wd_ref[0].astype(jnp.bfloat16)
        g = lax.dot_general(xb, wg, (((1,), (1,)), ((), ())),
                            preferred_element_type=jnp.float32)
        u = lax.dot_general(xb, wu, (((1,), (1,)), ((), ())),
                            preferred_element_type=jnp.float32)
        h = g * jax.lax.logistic(g) * u
        y = lax.dot_general(h.astype(jnp.bfloat16), wd,
                            (((1,), (1,)), ((), ())),
                            preferred_element_type=jnp.float32)
        out_ref[...] = y * wrow_ref[...][:, 0:1]


def _mlp(xs, wrow, Wg, Wu, Wd, bexpert, bvalid, interpret=False):
    grid_spec = pltpu.PrefetchScalarGridSpec(
        num_scalar_prefetch=2,
        grid=(NB,),
        in_specs=[
            pl.BlockSpec((BM, D), lambda b, be, bv: (b, 0)),
            pl.BlockSpec((BM, LANES), lambda b, be, bv: (b, 0)),
            pl.BlockSpec((1, FF, D), lambda b, be, bv: (be[b], 0, 0)),
            pl.BlockSpec((1, FF, D), lambda b, be, bv: (be[b], 0, 0)),
            pl.BlockSpec((1, D, FF), lambda b, be, bv: (be[b], 0, 0)),
        ],
        out_specs=pl.BlockSpec((BM, D), lambda b, be, bv: (b, 0)),
    )
    return pl.pallas_call(
        _mlp_body,
        grid_spec=grid_spec,
        out_shape=jax.ShapeDtypeStruct((P, D), jnp.float32),
        interpret=interpret,
    )(bexpert, bvalid, xs, wrow, Wg, Wu, Wd)


CH_D = 32


def _dispatch_body(x_hbm, r1_hbm, r2_hbm, w1r_hbm, w2r_hbm, xs_hbm, wrow_hbm,
                   idx1_v, idx2_v, rows_v, w1r_v, w2r_v, sem):
    wid = lax.axis_index("s") * NC + lax.axis_index("c")
    tpw = T // NW
    for it in range(tpw // CH_D):
        base = wid * tpw + it * CH_D
        pltpu.sync_copy(x_hbm.at[pl.ds(base, CH_D)], rows_v)
        pltpu.sync_copy(r1_hbm.at[pl.ds(base, CH_D)], idx1_v)
        pltpu.sync_copy(r2_hbm.at[pl.ds(base, CH_D)], idx2_v)
        pltpu.sync_copy(w1r_hbm.at[pl.ds(base, CH_D)], w1r_v)
        pltpu.sync_copy(w2r_hbm.at[pl.ds(base, CH_D)], w2r_v)
        c1 = pltpu.async_copy(rows_v, xs_hbm.at[idx1_v], sem)
        c2 = pltpu.async_copy(rows_v, xs_hbm.at[idx2_v], sem)
        c3 = pltpu.async_copy(w1r_v, wrow_hbm.at[idx1_v], sem)
        c4 = pltpu.async_copy(w2r_v, wrow_hbm.at[idx2_v], sem)
        c1.wait()
        c2.wait()
        c3.wait()
        c4.wait()


def _dispatch(x, r1, r2, w1rep, w2rep):
    return pl.kernel(
        _dispatch_body,
        out_type=(
            jax.ShapeDtypeStruct((P, D), jnp.float32),
            jax.ShapeDtypeStruct((P, LANES), jnp.float32),
        ),
        mesh=plsc.VectorSubcoreMesh(**_SC_MESH),
        scratch_types=[
            pltpu.VMEM((CH_D,), jnp.int32),
            pltpu.VMEM((CH_D,), jnp.int32),
            pltpu.VMEM((CH_D, D), jnp.float32),
            pltpu.VMEM((CH_D, LANES), jnp.float32),
            pltpu.VMEM((CH_D, LANES), jnp.float32),
            pltpu.SemaphoreType.DMA,
        ],
    )(x, r1, r2, w1rep, w2rep)


CH_C = 16


def _combine_body(ys_hbm, r1_hbm, r2_hbm, out_hbm,
                  idx1_v, idx2_v, buf1_v, buf2_v, sem):
    wid = lax.axis_index("s") * NC + lax.axis_index("c")
    tpw = T // NW
    for it in range(tpw // CH_C):
        base = wid * tpw + it * CH_C
        pltpu.sync_copy(r1_hbm.at[pl.ds(base, CH_C)], idx1_v)
        pltpu.sync_copy(r2_hbm.at[pl.ds(base, CH_C)], idx2_v)
        c1 = pltpu.async_copy(ys_hbm.at[idx1_v], buf1_v, sem)
        c2 = pltpu.async_copy(ys_hbm.at[idx2_v], buf2_v, sem)
        c1.wait()
        c2.wait()

        def row_body(i, _):
            for j in range(D // 16):
                a = buf1_v[i, pl.ds(j * 16, 16)]
                b = buf2_v[i, pl.ds(j * 16, 16)]
                buf1_v[i, pl.ds(j * 16, 16)] = a + b
            return 0

        lax.fori_loop(0, CH_C, row_body, 0)
        pltpu.sync_copy(buf1_v, out_hbm.at[pl.ds(base, CH_C)])


def _combine(ys, r1, r2):
    return pl.kernel(
        _combine_body,
        out_type=jax.ShapeDtypeStruct((T, D), jnp.float32),
        mesh=plsc.VectorSubcoreMesh(**_SC_MESH),
        scratch_types=[
            pltpu.VMEM((CH_C,), jnp.int32),
            pltpu.VMEM((CH_C,), jnp.int32),
            pltpu.VMEM((CH_C, D), jnp.float32),
            pltpu.VMEM((CH_C, D), jnp.float32),
            pltpu.SemaphoreType.DMA,
        ],
    )(ys, r1, r2)


def _block_map(counts8):
    pc = ((counts8 + (BM - 1)) // BM) * BM
    ends = jnp.cumsum(pc) // BM
    b = jnp.arange(NB, dtype=jnp.int32)
    bexpert = jnp.minimum(
        jnp.sum(ends[None, :] <= b[:, None], axis=1).astype(jnp.int32), E - 1)
    bvalid = (b < ends[-1]).astype(jnp.int32)
    return bexpert, bvalid


def kernel(hidden_states, gate_w, Wg, Wu, Wd):
    bsz, seq, d = hidden_states.shape
    x = hidden_states.reshape(-1, d)
    gwp = jnp.zeros((LANES, D), jnp.float32).at[:E].set(gate_w)
    idx, w1rep, w2rep, cnt = _router(x, gwp)
    r1 = idx[:, 0]
    r2 = idx[:, 1]
    bexpert, bvalid = _block_map(cnt[0, :E])
    xs, wrow = _dispatch(x, r1, r2, w1rep, w2rep)
    ys = _mlp(xs, wrow, Wg, Wu, Wd, bexpert, bvalid)
    out = _combine(ys, r1, r2)
    return out.reshape(bsz, seq, d)
